# Initial kernel scaffold; baseline (speedup 1.0000x reference)
import functools

import jax
import jax.numpy as jnp
from jax import lax
from jax.experimental import pallas as pl
from jax.experimental.pallas import tpu as pltpu
from jax.experimental.pallas import tpu_sc as plsc

N_U = 25000
N_E = 25000
N = N_U + N_E
D = 32
E = 800000
NC, NS = 2, 16
NW = NC * NS
CH = 128
NFULL = 195
EPT = NFULL * CH
REM_BASE = EPT * NW
REM_CHUNKS = (E - REM_BASE) // CH
ROWS_PER_TILE = N // NS
ZR = 625
MM_BLK = 2000


def _xw_body(x_ref, w_ref, o_ref):
    o_ref[...] = jnp.dot(x_ref[...], w_ref[...],
                         preferred_element_type=jnp.float32)


def _transform(tab, w):
    return pl.pallas_call(
        _xw_body,
        grid=(N // MM_BLK,),
        in_specs=[
            pl.BlockSpec((MM_BLK, D), lambda i: (i, 0)),
            pl.BlockSpec((D, D), lambda i: (0, 0)),
        ],
        out_specs=pl.BlockSpec((MM_BLK, D), lambda i: (i, 0)),
        out_shape=jax.ShapeDtypeStruct((N, D), jnp.float32),
    )(tab, w)


def _sc_edge_pass(t, idx, src, dst):
    mesh = plsc.VectorSubcoreMesh(core_axis_name="c", subcore_axis_name="s",
                                  num_cores=NC, num_subcores=NS)

    @functools.partial(
        pl.kernel,
        out_type=(
            jax.ShapeDtypeStruct((NC * N, D), jnp.float32),
            jax.ShapeDtypeStruct((NW, 16), jnp.float32),
        ),
        mesh=mesh,
        scratch_types=[
            pltpu.VMEM_SHARED((N, D), jnp.float32),
            pltpu.VMEM((N,), jnp.int32),
            pltpu.VMEM((CH,), jnp.int32),
            pltpu.VMEM((CH,), jnp.int32),
            pltpu.VMEM((CH,), jnp.int32),
            pltpu.VMEM((CH,), jnp.int32),
            pltpu.VMEM((CH, D), jnp.float32),
            pltpu.VMEM((CH, D), jnp.float32),
            pltpu.VMEM((CH, D), jnp.float32),
            pltpu.VMEM((ZR, D), jnp.float32),
            pltpu.VMEM((16,), jnp.float32),
            pltpu.SemaphoreType.DMA,
            pltpu.SemaphoreType.DMA,
        ],
    )
    def k(t_hbm, idx_hbm, src_hbm, dst_hbm, acc_hbm, z_hbm,
          acc_sh, idx_v, src_v, dst_v, csrc_v, cdst_v, srow_v, drow_v,
          msg_v, zero_v, z_v, sem_a, sem_b):
        c = lax.axis_index("c")
        s = lax.axis_index("s")
        gwid = c * NS + s

        zeros16 = jnp.zeros((16,), jnp.float32)
        pltpu.sync_copy(idx_hbm, idx_v)

        @pl.loop(0, ZR)
        def _(i):
            zero_v[i, pl.ds(0, 16)] = zeros16
            zero_v[i, pl.ds(16, 16)] = zeros16

        z_v[...] = zeros16
        row0 = s * ROWS_PER_TILE
        for r in range(ROWS_PER_TILE // ZR):
            pltpu.sync_copy(zero_v, acc_sh.at[pl.ds(row0 + r * ZR, ZR)])
        plsc.subcore_barrier()

        lane = lax.iota(jnp.int32, 16)

        def chunk(base):
            pltpu.sync_copy(src_hbm.at[pl.ds(base, CH)], src_v)
            pltpu.sync_copy(dst_hbm.at[pl.ds(base, CH)], dst_v)
            for g in range(CH // 16):
                sg = src_v[pl.ds(g * 16, 16)]
                dg = dst_v[pl.ds(g * 16, 16)]
                csrc_v[pl.ds(g * 16, 16)] = plsc.load_gather(idx_v, [sg])
                cdst_v[pl.ds(g * 16, 16)] = plsc.load_gather(idx_v, [dg])
            cp_a = pltpu.async_copy(t_hbm.at[csrc_v], srow_v, sem_a)
            cp_b = pltpu.async_copy(t_hbm.at[cdst_v], drow_v, sem_b)
            cp_a.wait()
            cp_b.wait()
            for g in range(CH // 16):
                eids = lane + (g * 16)
                acc = jnp.zeros((16,), jnp.float32)
                for d in range(D):
                    dsp = jnp.full((16,), d, jnp.int32)
                    es = plsc.load_gather(srow_v, [eids, dsp])
                    ed = plsc.load_gather(drow_v, [eids, dsp])
                    acc = acc + es * ed
                w = jnp.exp(jnp.maximum(acc, 0.2 * acc))
                z_v[...] = z_v[...] + w
                for d in range(D):
                    dsp = jnp.full((16,), d, jnp.int32)
                    es = plsc.load_gather(srow_v, [eids, dsp])
                    plsc.store_scatter(msg_v, [eids, dsp], w * es)
            pltpu.sync_copy(msg_v, acc_sh.at[dst_v], add=True)

        tile_base = gwid * EPT

        @pl.loop(0, NFULL)
        def _(j):
            chunk(pl.multiple_of(tile_base + j * CH, 8))

        @pl.when(gwid < REM_CHUNKS)
        def _():
            chunk(pl.multiple_of(REM_BASE + gwid * CH, 8))

        plsc.subcore_barrier()
        out_base = c * N + row0
        pltpu.sync_copy(acc_sh.at[pl.ds(row0, ROWS_PER_TILE)],
                        acc_hbm.at[pl.ds(out_base, ROWS_PER_TILE)])
        pltpu.sync_copy(z_v, z_hbm.at[gwid])

    return k(t, idx, src, dst)


def _combine_body(a0_ref, a1_ref, z_ref, o_ref):
    zsum = jnp.sum(z_ref[...])
    o_ref[...] = jnp.maximum((a0_ref[...] + a1_ref[...]) / zsum, 0.0)


def _combine(acc, z):
    nblk = N // MM_BLK
    return pl.pallas_call(
        _combine_body,
        grid=(nblk,),
        in_specs=[
            pl.BlockSpec((MM_BLK, D), lambda i: (i, 0)),
            pl.BlockSpec((MM_BLK, D), lambda i, _n=nblk: (i + _n, 0)),
            pl.BlockSpec((NW, 16), lambda i: (0, 0)),
        ],
        out_specs=pl.BlockSpec((MM_BLK, D), lambda i: (i, 0)),
        out_shape=jax.ShapeDtypeStruct((N, D), jnp.float32),
    )(acc, acc, z)


def kernel(user_emb, entity_emb, W, W_r, user_indices, item_indices,
           edge_index_ui, edge_index_kg, edge_type_kg):
    tab = jnp.concatenate([user_emb, entity_emb], axis=0)
    t = _transform(tab, W)
    idx = jnp.concatenate([user_indices.astype(jnp.int32),
                           item_indices.astype(jnp.int32) + N_U])
    src = edge_index_ui[0].astype(jnp.int32)
    dst = edge_index_ui[1].astype(jnp.int32)
    acc, z = _sc_edge_pass(t, idx, src, dst)
    x = _combine(acc, z)
    return (x[:N_U], x[N_U:])


# speedup vs baseline: 2.5171x; 2.5171x over previous
import dataclasses
import functools

import jax
import jax.numpy as jnp
from jax import lax
from jax.experimental import pallas as pl
from jax.experimental.pallas import tpu as pltpu
from jax.experimental.pallas import tpu_sc as plsc

N_U = 25000
N_E = 25000
N = N_U + N_E
D = 32
E = 800000
NC, NS = 2, 16
NW = NC * NS
CH = 128
NFULL = 195
EPT = NFULL * CH
REM_BASE = EPT * NW
REM_CHUNKS = (E - REM_BASE) // CH
GFULL = 390
G_ROUNDS = GFULL // NW
G_EXTRA = GFULL - G_ROUNDS * NW
G_TAIL_BASE = GFULL * CH
G_TAIL = N - G_TAIL_BASE
ROWS_BIG = 3128
ROWS_SMALL = 3120
BIG_TILES = 10
ZR = 128
MM_BLK = 2000


def _sc_compiler_params():
    cp = pltpu.CompilerParams()
    fields = pltpu.CompilerParams.__dataclass_fields__
    if "needs_layout_passes" in fields:
        cp = dataclasses.replace(cp, needs_layout_passes=False)
    if "use_tc_tiling_on_sc" in fields:
        cp = dataclasses.replace(cp, use_tc_tiling_on_sc=False)
    return cp


def _sc_mesh():
    return plsc.VectorSubcoreMesh(core_axis_name="c", subcore_axis_name="s",
                                  num_cores=NC, num_subcores=NS)


def _xw_body(x_ref, w_ref, o_ref):
    o_ref[...] = jnp.dot(x_ref[...], w_ref[...],
                         preferred_element_type=jnp.float32)


def _transform(tab, w):
    return pl.pallas_call(
        _xw_body,
        grid=(N // MM_BLK,),
        in_specs=[
            pl.BlockSpec((MM_BLK, D), lambda i: (i, 0)),
            pl.BlockSpec((D, D), lambda i: (0, 0)),
        ],
        out_specs=pl.BlockSpec((MM_BLK, D), lambda i: (i, 0)),
        out_shape=jax.ShapeDtypeStruct((N, D), jnp.float32),
    )(tab, w)


def _sc_node_gather(t, idx):
    @functools.partial(
        pl.kernel,
        compiler_params=_sc_compiler_params(),
        out_type=jax.ShapeDtypeStruct((N, D), jnp.float32),
        mesh=_sc_mesh(),
        scratch_types=[
            pltpu.VMEM((CH,), jnp.int32),
            pltpu.VMEM((CH, D), jnp.float32),
            pltpu.VMEM((G_TAIL,), jnp.int32),
            pltpu.VMEM((G_TAIL, D), jnp.float32),
            pltpu.SemaphoreType.DMA,
        ],
    )
    def k(t_hbm, idx_hbm, y_hbm, cidx_v, rows_v, cidx_t, rows_t, sem):
        c = lax.axis_index("c")
        s = lax.axis_index("s")
        gwid = c * NS + s

        def chunk(base):
            pltpu.sync_copy(idx_hbm.at[pl.ds(base, CH)], cidx_v)
            pltpu.async_copy(t_hbm.at[cidx_v], rows_v, sem).wait()
            pltpu.sync_copy(rows_v, y_hbm.at[pl.ds(base, CH)])

        @pl.loop(0, G_ROUNDS)
        def _(j):
            chunk(pl.multiple_of((j * NW + gwid) * CH, 8))

        @pl.when(gwid < G_EXTRA)
        def _():
            chunk(pl.multiple_of((G_ROUNDS * NW + gwid) * CH, 8))

        @pl.when(gwid == G_EXTRA)
        def _():
            base = pl.multiple_of(G_TAIL_BASE, 8)
            pltpu.sync_copy(idx_hbm.at[pl.ds(base, G_TAIL)], cidx_t)
            pltpu.async_copy(t_hbm.at[cidx_t], rows_t, sem).wait()
            pltpu.sync_copy(rows_t, y_hbm.at[pl.ds(base, G_TAIL)])

    return k(t, idx)


def _sc_edge_pass(y, src, dst):
    @functools.partial(
        pl.kernel,
        compiler_params=_sc_compiler_params(),
        out_type=(
            jax.ShapeDtypeStruct((NC * N, D), jnp.float32),
            jax.ShapeDtypeStruct((NW * 16,), jnp.float32),
        ),
        mesh=_sc_mesh(),
        scratch_types=[
            pltpu.VMEM_SHARED((N, D), jnp.float32),
            pltpu.VMEM((CH,), jnp.int32),
            pltpu.VMEM((CH,), jnp.int32),
            pltpu.VMEM((CH, D), jnp.float32),
            pltpu.VMEM((CH, D), jnp.float32),
            pltpu.VMEM((CH, D), jnp.float32),
            pltpu.VMEM((ZR, D), jnp.float32),
            pltpu.VMEM((16,), jnp.float32),
            pltpu.SemaphoreType.DMA,
            pltpu.SemaphoreType.DMA,
        ],
    )
    def k(y_hbm, src_hbm, dst_hbm, acc_hbm, z_hbm,
          acc_sh, src_v, dst_v, srow_v, drow_v, msg_v, zero_v, z_v,
          sem_a, sem_b):
        c = lax.axis_index("c")
        s = lax.axis_index("s")
        gwid = c * NS + s

        zeros16 = jnp.zeros((16,), jnp.float32)

        @pl.loop(0, ZR)
        def _(i):
            zero_v[i, pl.ds(0, 16)] = zeros16
            zero_v[i, pl.ds(16, 16)] = zeros16

        z_v[...] = zeros16

        def zero_rows(start_row, nrows):
            for off in range(0, nrows, ZR):
                sz = min(ZR, nrows - off)
                pltpu.sync_copy(zero_v.at[pl.ds(0, sz)],
                                acc_sh.at[pl.ds(start_row + off, sz)])

        @pl.when(s < BIG_TILES)
        def _():
            zero_rows(pl.multiple_of(s * ROWS_BIG, 8), ROWS_BIG)

        @pl.when(s >= BIG_TILES)
        def _():
            zero_rows(pl.multiple_of(
                BIG_TILES * ROWS_BIG + (s - BIG_TILES) * ROWS_SMALL, 8),
                ROWS_SMALL)

        plsc.subcore_barrier()

        lane = lax.iota(jnp.int32, 16)

        def chunk(base):
            pltpu.sync_copy(src_hbm.at[pl.ds(base, CH)], src_v)
            pltpu.sync_copy(dst_hbm.at[pl.ds(base, CH)], dst_v)
            cp_a = pltpu.async_copy(y_hbm.at[src_v], srow_v, sem_a)
            cp_b = pltpu.async_copy(y_hbm.at[dst_v], drow_v, sem_b)
            cp_a.wait()
            cp_b.wait()
            for g in range(CH // 16):
                eids = lane + (g * 16)
                acc = jnp.zeros((16,), jnp.float32)
                for d in range(D):
                    dsp = jnp.full((16,), d, jnp.int32)
                    es = plsc.load_gather(srow_v, [eids, dsp])
                    ed = plsc.load_gather(drow_v, [eids, dsp])
                    acc = acc + es * ed
                w = jnp.exp(jnp.maximum(acc, 0.2 * acc))
                z_v[...] = z_v[...] + w
                for d in range(D):
                    dsp = jnp.full((16,), d, jnp.int32)
                    es = plsc.load_gather(srow_v, [eids, dsp])
                    plsc.store_scatter(msg_v, [eids, dsp], w * es)
            pltpu.sync_copy(msg_v, acc_sh.at[dst_v], add=True)

        tile_base = gwid * EPT

        @pl.loop(0, NFULL)
        def _(j):
            chunk(pl.multiple_of(tile_base + j * CH, 8))

        @pl.when(gwid < REM_CHUNKS)
        def _():
            chunk(pl.multiple_of(REM_BASE + gwid * CH, 8))

        plsc.subcore_barrier()

        @pl.when(s < BIG_TILES)
        def _():
            rs = pl.multiple_of(s * ROWS_BIG, 8)
            pltpu.sync_copy(acc_sh.at[pl.ds(rs, ROWS_BIG)],
                            acc_hbm.at[pl.ds(c * N + rs, ROWS_BIG)])

        @pl.when(s >= BIG_TILES)
        def _():
            rs = pl.multiple_of(
                BIG_TILES * ROWS_BIG + (s - BIG_TILES) * ROWS_SMALL, 8)
            pltpu.sync_copy(acc_sh.at[pl.ds(rs, ROWS_SMALL)],
                            acc_hbm.at[pl.ds(c * N + rs, ROWS_SMALL)])

        pltpu.sync_copy(z_v, z_hbm.at[pl.ds(pl.multiple_of(gwid * 16, 8), 16)])

    return k(y, src, dst)


def _combine_body(a0_ref, a1_ref, z_ref, o_ref):
    zsum = jnp.sum(z_ref[...])
    o_ref[...] = jnp.maximum((a0_ref[...] + a1_ref[...]) / zsum, 0.0)


def _combine(acc, z):
    nblk = N // MM_BLK
    return pl.pallas_call(
        _combine_body,
        grid=(nblk,),
        in_specs=[
            pl.BlockSpec((MM_BLK, D), lambda i: (i, 0)),
            pl.BlockSpec((MM_BLK, D), lambda i, _n=nblk: (i + _n, 0)),
            pl.BlockSpec((NW, 16), lambda i: (0, 0)),
        ],
        out_specs=pl.BlockSpec((MM_BLK, D), lambda i: (i, 0)),
        out_shape=jax.ShapeDtypeStruct((N, D), jnp.float32),
    )(acc, acc, z)


def kernel(user_emb, entity_emb, W, W_r, user_indices, item_indices,
           edge_index_ui, edge_index_kg, edge_type_kg):
    tab = jnp.concatenate([user_emb, entity_emb], axis=0)
    t = _transform(tab, W)
    idx = jnp.concatenate([user_indices.astype(jnp.int32),
                           item_indices.astype(jnp.int32) + N_U])
    src = edge_index_ui[0].astype(jnp.int32)
    dst = edge_index_ui[1].astype(jnp.int32)
    y = _sc_node_gather(t, idx)
    acc, z = _sc_edge_pass(y, src, dst)
    x = _combine(acc, z.reshape(NW, 16))
    return (x[:N_U], x[N_U:])


# speedup vs baseline: 2.6964x; 1.0713x over previous
import dataclasses
import functools

import jax
import jax.numpy as jnp
from jax import lax
from jax.experimental import pallas as pl
from jax.experimental.pallas import tpu as pltpu
from jax.experimental.pallas import tpu_sc as plsc

N_U = 25000
N_E = 25000
N = N_U + N_E
D = 32
E = 800000
NC, NS = 2, 16
NW = NC * NS
CH = 128
NFULL = 195
EPT = NFULL * CH
REM_BASE = EPT * NW
REM_CHUNKS = (E - REM_BASE) // CH
GFULL = 390
G_ROUNDS = GFULL // NW
G_EXTRA = GFULL - G_ROUNDS * NW
G_TAIL_BASE = GFULL * CH
G_TAIL = N - G_TAIL_BASE
ROWS_BIG = 3128
ROWS_SMALL = 3120
BIG_TILES = 10
ZR = 128
MM_BLK = 2000


def _sc_compiler_params():
    cp = pltpu.CompilerParams()
    fields = pltpu.CompilerParams.__dataclass_fields__
    if "needs_layout_passes" in fields:
        cp = dataclasses.replace(cp, needs_layout_passes=False)
    if "use_tc_tiling_on_sc" in fields:
        cp = dataclasses.replace(cp, use_tc_tiling_on_sc=False)
    return cp


def _sc_mesh():
    return plsc.VectorSubcoreMesh(core_axis_name="c", subcore_axis_name="s",
                                  num_cores=NC, num_subcores=NS)


def _xw_body(x_ref, w_ref, o_ref):
    o_ref[...] = jnp.dot(x_ref[...], w_ref[...],
                         preferred_element_type=jnp.float32)


def _transform(tab, w):
    return pl.pallas_call(
        _xw_body,
        grid=(N // MM_BLK,),
        in_specs=[
            pl.BlockSpec((MM_BLK, D), lambda i: (i, 0)),
            pl.BlockSpec((D, D), lambda i: (0, 0)),
        ],
        out_specs=pl.BlockSpec((MM_BLK, D), lambda i: (i, 0)),
        out_shape=jax.ShapeDtypeStruct((N, D), jnp.float32),
    )(tab, w)


def _sc_node_gather(t, idx):
    @functools.partial(
        pl.kernel,
        compiler_params=_sc_compiler_params(),
        out_type=jax.ShapeDtypeStruct((N, D), jnp.float32),
        mesh=_sc_mesh(),
        scratch_types=[
            pltpu.VMEM((CH,), jnp.int32),
            pltpu.VMEM((CH, D), jnp.float32),
            pltpu.VMEM((G_TAIL,), jnp.int32),
            pltpu.VMEM((G_TAIL, D), jnp.float32),
            pltpu.SemaphoreType.DMA,
        ],
    )
    def k(t_hbm, idx_hbm, y_hbm, cidx_v, rows_v, cidx_t, rows_t, sem):
        c = lax.axis_index("c")
        s = lax.axis_index("s")
        gwid = c * NS + s

        def chunk(base):
            pltpu.sync_copy(idx_hbm.at[pl.ds(base, CH)], cidx_v)
            pltpu.async_copy(t_hbm.at[cidx_v], rows_v, sem).wait()
            pltpu.sync_copy(rows_v, y_hbm.at[pl.ds(base, CH)])

        @pl.loop(0, G_ROUNDS)
        def _(j):
            chunk(pl.multiple_of((j * NW + gwid) * CH, 8))

        @pl.when(gwid < G_EXTRA)
        def _():
            chunk(pl.multiple_of((G_ROUNDS * NW + gwid) * CH, 8))

        @pl.when(gwid == G_EXTRA)
        def _():
            base = pl.multiple_of(G_TAIL_BASE, 8)
            pltpu.sync_copy(idx_hbm.at[pl.ds(base, G_TAIL)], cidx_t)
            pltpu.async_copy(t_hbm.at[cidx_t], rows_t, sem).wait()
            pltpu.sync_copy(rows_t, y_hbm.at[pl.ds(base, G_TAIL)])

    return k(t, idx)


def _sc_edge_pass(y, src, dst):
    @functools.partial(
        pl.kernel,
        compiler_params=_sc_compiler_params(),
        out_type=(
            jax.ShapeDtypeStruct((NC * N, D), jnp.float32),
            jax.ShapeDtypeStruct((NW * 16,), jnp.float32),
        ),
        mesh=_sc_mesh(),
        scratch_types=[
            pltpu.VMEM_SHARED((N, D), jnp.float32),
            pltpu.VMEM((CH,), jnp.int32),
            pltpu.VMEM((CH,), jnp.int32),
            pltpu.VMEM((CH,), jnp.int32),
            pltpu.VMEM((CH,), jnp.int32),
            pltpu.VMEM((CH, D), jnp.float32),
            pltpu.VMEM((CH, D), jnp.float32),
            pltpu.VMEM((CH, D), jnp.float32),
            pltpu.VMEM((CH, D), jnp.float32),
            pltpu.VMEM((CH, D), jnp.float32),
            pltpu.VMEM((ZR, D), jnp.float32),
            pltpu.VMEM((16,), jnp.float32),
            pltpu.SemaphoreType.DMA,
            pltpu.SemaphoreType.DMA,
        ],
    )
    def k(y_hbm, src_hbm, dst_hbm, acc_hbm, z_hbm,
          acc_sh, src_a, dst_a, src_b, dst_b, srow_a, drow_a, srow_b, drow_b,
          msg_v, zero_v, z_v, sem_a, sem_b):
        c = lax.axis_index("c")
        s = lax.axis_index("s")
        gwid = c * NS + s

        zeros16 = jnp.zeros((16,), jnp.float32)

        @pl.loop(0, ZR)
        def _(i):
            zero_v[i, pl.ds(0, 16)] = zeros16
            zero_v[i, pl.ds(16, 16)] = zeros16

        z_v[...] = zeros16

        def zero_rows(start_row, nrows):
            for off in range(0, nrows, ZR):
                sz = min(ZR, nrows - off)
                pltpu.sync_copy(zero_v.at[pl.ds(0, sz)],
                                acc_sh.at[pl.ds(start_row + off, sz)])

        @pl.when(s < BIG_TILES)
        def _():
            zero_rows(pl.multiple_of(s * ROWS_BIG, 8), ROWS_BIG)

        @pl.when(s >= BIG_TILES)
        def _():
            zero_rows(pl.multiple_of(
                BIG_TILES * ROWS_BIG + (s - BIG_TILES) * ROWS_SMALL, 8),
                ROWS_SMALL)

        plsc.subcore_barrier()

        lane = lax.iota(jnp.int32, 16)

        def ids_load(sv, dv, base):
            pltpu.sync_copy(src_hbm.at[pl.ds(base, CH)], sv)
            pltpu.sync_copy(dst_hbm.at[pl.ds(base, CH)], dv)

        def rows_start(sv, dv, sr, dr, sem):
            pltpu.async_copy(y_hbm.at[sv], sr, sem)
            pltpu.async_copy(y_hbm.at[dv], dr, sem)

        def rows_wait(sv, dv, sr, dr, sem):
            pltpu.make_async_copy(y_hbm.at[sv], sr, sem).wait()
            pltpu.make_async_copy(y_hbm.at[dv], dr, sem).wait()

        def compute_scatter(sr, dr, dv):
            for g in range(CH // 16):
                eids = lane + (g * 16)
                acc = jnp.zeros((16,), jnp.float32)
                for d in range(D):
                    dsp = jnp.full((16,), d, jnp.int32)
                    es = plsc.load_gather(sr, [eids, dsp])
                    ed = plsc.load_gather(dr, [eids, dsp])
                    acc = acc + es * ed
                w = jnp.exp(jnp.maximum(acc, 0.2 * acc))
                z_v[...] = z_v[...] + w
                for d in range(D):
                    dsp = jnp.full((16,), d, jnp.int32)
                    es = plsc.load_gather(sr, [eids, dsp])
                    plsc.store_scatter(msg_v, [eids, dsp], w * es)
            pltpu.sync_copy(msg_v, acc_sh.at[dv], add=True)

        tile_base = gwid * EPT

        ids_load(src_a, dst_a, pl.multiple_of(tile_base, 8))
        rows_start(src_a, dst_a, srow_a, drow_a, sem_a)

        @pl.loop(0, (NFULL - 1) // 2)
        def _(p):
            b1 = pl.multiple_of(tile_base + (2 * p + 1) * CH, 8)
            ids_load(src_b, dst_b, b1)
            rows_start(src_b, dst_b, srow_b, drow_b, sem_b)
            rows_wait(src_a, dst_a, srow_a, drow_a, sem_a)
            compute_scatter(srow_a, drow_a, dst_a)
            b2 = pl.multiple_of(tile_base + (2 * p + 2) * CH, 8)
            ids_load(src_a, dst_a, b2)
            rows_start(src_a, dst_a, srow_a, drow_a, sem_a)
            rows_wait(src_b, dst_b, srow_b, drow_b, sem_b)
            compute_scatter(srow_b, drow_b, dst_b)

        rows_wait(src_a, dst_a, srow_a, drow_a, sem_a)
        compute_scatter(srow_a, drow_a, dst_a)

        @pl.when(gwid < REM_CHUNKS)
        def _():
            base = pl.multiple_of(REM_BASE + gwid * CH, 8)
            ids_load(src_b, dst_b, base)
            rows_start(src_b, dst_b, srow_b, drow_b, sem_b)
            rows_wait(src_b, dst_b, srow_b, drow_b, sem_b)
            compute_scatter(srow_b, drow_b, dst_b)

        plsc.subcore_barrier()

        @pl.when(s < BIG_TILES)
        def _():
            rs = pl.multiple_of(s * ROWS_BIG, 8)
            pltpu.sync_copy(acc_sh.at[pl.ds(rs, ROWS_BIG)],
                            acc_hbm.at[pl.ds(c * N + rs, ROWS_BIG)])

        @pl.when(s >= BIG_TILES)
        def _():
            rs = pl.multiple_of(
                BIG_TILES * ROWS_BIG + (s - BIG_TILES) * ROWS_SMALL, 8)
            pltpu.sync_copy(acc_sh.at[pl.ds(rs, ROWS_SMALL)],
                            acc_hbm.at[pl.ds(c * N + rs, ROWS_SMALL)])

        pltpu.sync_copy(z_v, z_hbm.at[pl.ds(pl.multiple_of(gwid * 16, 8), 16)])

    return k(y, src, dst)


def _combine_body(a0_ref, a1_ref, z_ref, o_ref):
    zsum = jnp.sum(z_ref[...])
    o_ref[...] = jnp.maximum((a0_ref[...] + a1_ref[...]) / zsum, 0.0)


def _combine(acc, z):
    nblk = N // MM_BLK
    return pl.pallas_call(
        _combine_body,
        grid=(nblk,),
        in_specs=[
            pl.BlockSpec((MM_BLK, D), lambda i: (i, 0)),
            pl.BlockSpec((MM_BLK, D), lambda i, _n=nblk: (i + _n, 0)),
            pl.BlockSpec((NW, 16), lambda i: (0, 0)),
        ],
        out_specs=pl.BlockSpec((MM_BLK, D), lambda i: (i, 0)),
        out_shape=jax.ShapeDtypeStruct((N, D), jnp.float32),
    )(acc, acc, z)


def kernel(user_emb, entity_emb, W, W_r, user_indices, item_indices,
           edge_index_ui, edge_index_kg, edge_type_kg):
    tab = jnp.concatenate([user_emb, entity_emb], axis=0)
    t = _transform(tab, W)
    idx = jnp.concatenate([user_indices.astype(jnp.int32),
                           item_indices.astype(jnp.int32) + N_U])
    src = edge_index_ui[0].astype(jnp.int32)
    dst = edge_index_ui[1].astype(jnp.int32)
    y = _sc_node_gather(t, idx)
    acc, z = _sc_edge_pass(y, src, dst)
    x = _combine(acc, z.reshape(NW, 16))
    return (x[:N_U], x[N_U:])


# speedup vs baseline: 2.7074x; 1.0041x over previous
import dataclasses
import functools

import jax
import jax.numpy as jnp
from jax import lax
from jax.experimental import pallas as pl
from jax.experimental.pallas import tpu as pltpu
from jax.experimental.pallas import tpu_sc as plsc

N_U = 25000
N_E = 25000
N = N_U + N_E
D = 32
E = 800000
NC, NS = 2, 16
NW = NC * NS
CH = 128
NFULL = 195
EPT = NFULL * CH
REM_BASE = EPT * NW
REM_CHUNKS = (E - REM_BASE) // CH
GFULL = 390
G_ROUNDS = GFULL // NW
G_EXTRA = GFULL - G_ROUNDS * NW
G_TAIL_BASE = GFULL * CH
G_TAIL = N - G_TAIL_BASE
ROWS_BIG = 3128
ROWS_SMALL = 3120
BIG_TILES = 10
ZR = 128
MM_BLK = 2000


def _sc_compiler_params():
    cp = pltpu.CompilerParams()
    fields = pltpu.CompilerParams.__dataclass_fields__
    if "needs_layout_passes" in fields:
        cp = dataclasses.replace(cp, needs_layout_passes=False)
    if "use_tc_tiling_on_sc" in fields:
        cp = dataclasses.replace(cp, use_tc_tiling_on_sc=False)
    return cp


def _sc_mesh():
    return plsc.VectorSubcoreMesh(core_axis_name="c", subcore_axis_name="s",
                                  num_cores=NC, num_subcores=NS)


def _xw_body(x_ref, w_ref, o_ref):
    o_ref[...] = jnp.dot(x_ref[...], w_ref[...],
                         preferred_element_type=jnp.float32)


def _transform(tab, w):
    return pl.pallas_call(
        _xw_body,
        grid=(N // MM_BLK,),
        in_specs=[
            pl.BlockSpec((MM_BLK, D), lambda i: (i, 0)),
            pl.BlockSpec((D, D), lambda i: (0, 0)),
        ],
        out_specs=pl.BlockSpec((MM_BLK, D), lambda i: (i, 0)),
        out_shape=jax.ShapeDtypeStruct((N, D), jnp.float32),
    )(tab, w)


def _sc_node_gather(t, idx):
    @functools.partial(
        pl.kernel,
        compiler_params=_sc_compiler_params(),
        out_type=jax.ShapeDtypeStruct((N, D), jnp.float32),
        mesh=_sc_mesh(),
        scratch_types=[
            pltpu.VMEM((CH,), jnp.int32),
            pltpu.VMEM((CH, D), jnp.float32),
            pltpu.VMEM((G_TAIL,), jnp.int32),
            pltpu.VMEM((G_TAIL, D), jnp.float32),
            pltpu.SemaphoreType.DMA,
        ],
    )
    def k(t_hbm, idx_hbm, y_hbm, cidx_v, rows_v, cidx_t, rows_t, sem):
        c = lax.axis_index("c")
        s = lax.axis_index("s")
        gwid = c * NS + s

        def chunk(base):
            pltpu.sync_copy(idx_hbm.at[pl.ds(base, CH)], cidx_v)
            pltpu.async_copy(t_hbm.at[cidx_v], rows_v, sem).wait()
            pltpu.sync_copy(rows_v, y_hbm.at[pl.ds(base, CH)])

        @pl.loop(0, G_ROUNDS)
        def _(j):
            chunk(pl.multiple_of((j * NW + gwid) * CH, 8))

        @pl.when(gwid < G_EXTRA)
        def _():
            chunk(pl.multiple_of((G_ROUNDS * NW + gwid) * CH, 8))

        @pl.when(gwid == G_EXTRA)
        def _():
            base = pl.multiple_of(G_TAIL_BASE, 8)
            pltpu.sync_copy(idx_hbm.at[pl.ds(base, G_TAIL)], cidx_t)
            pltpu.async_copy(t_hbm.at[cidx_t], rows_t, sem).wait()
            pltpu.sync_copy(rows_t, y_hbm.at[pl.ds(base, G_TAIL)])

    return k(t, idx)


def _sc_edge_pass(y, src, dst):
    @functools.partial(
        pl.kernel,
        compiler_params=_sc_compiler_params(),
        out_type=(
            jax.ShapeDtypeStruct((NC * N, D), jnp.float32),
            jax.ShapeDtypeStruct((NW * 16,), jnp.float32),
        ),
        mesh=_sc_mesh(),
        scratch_types=[
            pltpu.VMEM_SHARED((N, D), jnp.float32),
            pltpu.VMEM((CH,), jnp.int32),
            pltpu.VMEM((CH,), jnp.int32),
            pltpu.VMEM((CH,), jnp.int32),
            pltpu.VMEM((CH,), jnp.int32),
            pltpu.VMEM((CH, D), jnp.float32),
            pltpu.VMEM((CH, D), jnp.float32),
            pltpu.VMEM((CH, D), jnp.float32),
            pltpu.VMEM((CH, D), jnp.float32),
            pltpu.VMEM((CH, D), jnp.float32),
            pltpu.VMEM((ZR, D), jnp.float32),
            pltpu.VMEM((16,), jnp.float32),
            pltpu.SemaphoreType.DMA,
            pltpu.SemaphoreType.DMA,
        ],
    )
    def k(y_hbm, src_hbm, dst_hbm, acc_hbm, z_hbm,
          acc_sh, src_a, dst_a, src_b, dst_b, srow_a, drow_a, srow_b, drow_b,
          msg_v, zero_v, z_v, sem_a, sem_b):
        c = lax.axis_index("c")
        s = lax.axis_index("s")
        gwid = c * NS + s

        zeros16 = jnp.zeros((16,), jnp.float32)

        @pl.loop(0, ZR)
        def _(i):
            zero_v[i, pl.ds(0, 16)] = zeros16
            zero_v[i, pl.ds(16, 16)] = zeros16

        z_v[...] = zeros16

        def zero_rows(start_row, nrows):
            for off in range(0, nrows, ZR):
                sz = min(ZR, nrows - off)
                pltpu.sync_copy(zero_v.at[pl.ds(0, sz)],
                                acc_sh.at[pl.ds(start_row + off, sz)])

        @pl.when(s < BIG_TILES)
        def _():
            zero_rows(pl.multiple_of(s * ROWS_BIG, 8), ROWS_BIG)

        @pl.when(s >= BIG_TILES)
        def _():
            zero_rows(pl.multiple_of(
                BIG_TILES * ROWS_BIG + (s - BIG_TILES) * ROWS_SMALL, 8),
                ROWS_SMALL)

        plsc.subcore_barrier()

        lane = lax.iota(jnp.int32, 16)

        def ids_load(sv, dv, base):
            pltpu.sync_copy(src_hbm.at[pl.ds(base, CH)], sv)
            pltpu.sync_copy(dst_hbm.at[pl.ds(base, CH)], dv)

        def rows_start(sv, dv, sr, dr, sem):
            pltpu.async_copy(y_hbm.at[sv], sr, sem)
            pltpu.async_copy(y_hbm.at[dv], dr, sem)

        def rows_wait(sv, dv, sr, dr, sem):
            pltpu.make_async_copy(y_hbm.at[sv], sr, sem).wait()
            pltpu.make_async_copy(y_hbm.at[dv], dr, sem).wait()

        def compute_scatter(sr, dr, dv):
            @pl.loop(0, CH // 16)
            def _(g):
                eids = lane + g * 16
                acc_a = jnp.zeros((16,), jnp.float32)
                acc_b = jnp.zeros((16,), jnp.float32)
                for d in range(D):
                    dsp = jnp.full((16,), d, jnp.int32)
                    es = plsc.load_gather(sr, [eids, dsp])
                    ed = plsc.load_gather(dr, [eids, dsp])
                    if d % 2 == 0:
                        acc_a = acc_a + es * ed
                    else:
                        acc_b = acc_b + es * ed
                l = acc_a + acc_b
                w = jnp.exp(jnp.maximum(l, 0.2 * l))
                for d in range(D):
                    dsp = jnp.full((16,), d, jnp.int32)
                    es = plsc.load_gather(sr, [eids, dsp])
                    plsc.store_scatter(msg_v, [eids, dsp], w * es)
                z_v[...] = z_v[...] + w

            pltpu.sync_copy(msg_v, acc_sh.at[dv], add=True)

        tile_base = gwid * EPT

        ids_load(src_a, dst_a, pl.multiple_of(tile_base, 8))
        rows_start(src_a, dst_a, srow_a, drow_a, sem_a)

        @pl.loop(0, (NFULL - 1) // 2)
        def _(p):
            b1 = pl.multiple_of(tile_base + (2 * p + 1) * CH, 8)
            ids_load(src_b, dst_b, b1)
            rows_start(src_b, dst_b, srow_b, drow_b, sem_b)
            rows_wait(src_a, dst_a, srow_a, drow_a, sem_a)
            compute_scatter(srow_a, drow_a, dst_a)
            b2 = pl.multiple_of(tile_base + (2 * p + 2) * CH, 8)
            ids_load(src_a, dst_a, b2)
            rows_start(src_a, dst_a, srow_a, drow_a, sem_a)
            rows_wait(src_b, dst_b, srow_b, drow_b, sem_b)
            compute_scatter(srow_b, drow_b, dst_b)

        rows_wait(src_a, dst_a, srow_a, drow_a, sem_a)
        compute_scatter(srow_a, drow_a, dst_a)

        @pl.when(gwid < REM_CHUNKS)
        def _():
            base = pl.multiple_of(REM_BASE + gwid * CH, 8)
            ids_load(src_b, dst_b, base)
            rows_start(src_b, dst_b, srow_b, drow_b, sem_b)
            rows_wait(src_b, dst_b, srow_b, drow_b, sem_b)
            compute_scatter(srow_b, drow_b, dst_b)

        plsc.subcore_barrier()

        @pl.when(s < BIG_TILES)
        def _():
            rs = pl.multiple_of(s * ROWS_BIG, 8)
            pltpu.sync_copy(acc_sh.at[pl.ds(rs, ROWS_BIG)],
                            acc_hbm.at[pl.ds(c * N + rs, ROWS_BIG)])

        @pl.when(s >= BIG_TILES)
        def _():
            rs = pl.multiple_of(
                BIG_TILES * ROWS_BIG + (s - BIG_TILES) * ROWS_SMALL, 8)
            pltpu.sync_copy(acc_sh.at[pl.ds(rs, ROWS_SMALL)],
                            acc_hbm.at[pl.ds(c * N + rs, ROWS_SMALL)])

        pltpu.sync_copy(z_v, z_hbm.at[pl.ds(pl.multiple_of(gwid * 16, 8), 16)])

    return k(y, src, dst)


def _combine_body(a0_ref, a1_ref, z_ref, o_ref):
    zsum = jnp.sum(z_ref[...])
    o_ref[...] = jnp.maximum((a0_ref[...] + a1_ref[...]) / zsum, 0.0)


def _combine(acc, z):
    nblk = N // MM_BLK
    return pl.pallas_call(
        _combine_body,
        grid=(nblk,),
        in_specs=[
            pl.BlockSpec((MM_BLK, D), lambda i: (i, 0)),
            pl.BlockSpec((MM_BLK, D), lambda i, _n=nblk: (i + _n, 0)),
            pl.BlockSpec((NW, 16), lambda i: (0, 0)),
        ],
        out_specs=pl.BlockSpec((MM_BLK, D), lambda i: (i, 0)),
        out_shape=jax.ShapeDtypeStruct((N, D), jnp.float32),
    )(acc, acc, z)


def kernel(user_emb, entity_emb, W, W_r, user_indices, item_indices,
           edge_index_ui, edge_index_kg, edge_type_kg):
    tab = jnp.concatenate([user_emb, entity_emb], axis=0)
    t = _transform(tab, W)
    idx = jnp.concatenate([user_indices.astype(jnp.int32),
                           item_indices.astype(jnp.int32) + N_U])
    src = edge_index_ui[0].astype(jnp.int32)
    dst = edge_index_ui[1].astype(jnp.int32)
    y = _sc_node_gather(t, idx)
    acc, z = _sc_edge_pass(y, src, dst)
    x = _combine(acc, z.reshape(NW, 16))
    return (x[:N_U], x[N_U:])


# speedup vs baseline: 10.0545x; 3.7136x over previous
import dataclasses
import functools

import jax
import jax.numpy as jnp
from jax import lax
from jax.experimental import pallas as pl
from jax.experimental.pallas import tpu as pltpu
from jax.experimental.pallas import tpu_sc as plsc

N_U = 25000
N_E = 25000
N = N_U + N_E
D = 32
E = 800000
NC, NS = 2, 16
NW = NC * NS
CH = 128
NFULL = 195
EPT = NFULL * CH
REM_BASE = EPT * NW
REM_CHUNKS = (E - REM_BASE) // CH
GFULL = 390
G_ROUNDS = GFULL // NW
G_EXTRA = GFULL - G_ROUNDS * NW
G_TAIL_BASE = GFULL * CH
G_TAIL = N - G_TAIL_BASE
ROWS_BIG = 3128
ROWS_SMALL = 3120
BIG_TILES = 10
ZR = 128
MM_BLK = 2000


def _sc_compiler_params():
    cp = pltpu.CompilerParams()
    fields = pltpu.CompilerParams.__dataclass_fields__
    if "needs_layout_passes" in fields:
        cp = dataclasses.replace(cp, needs_layout_passes=False)
    if "use_tc_tiling_on_sc" in fields:
        cp = dataclasses.replace(cp, use_tc_tiling_on_sc=False)
    return cp


def _sc_mesh():
    return plsc.VectorSubcoreMesh(core_axis_name="c", subcore_axis_name="s",
                                  num_cores=NC, num_subcores=NS)


def _xw_body(x_ref, w_ref, o_ref):
    o_ref[...] = jnp.dot(x_ref[...], w_ref[...],
                         preferred_element_type=jnp.float32)


def _transform(tab, w):
    return pl.pallas_call(
        _xw_body,
        grid=(N // MM_BLK,),
        in_specs=[
            pl.BlockSpec((MM_BLK, D), lambda i: (i, 0)),
            pl.BlockSpec((D, D), lambda i: (0, 0)),
        ],
        out_specs=pl.BlockSpec((MM_BLK, D), lambda i: (i, 0)),
        out_shape=jax.ShapeDtypeStruct((N, D), jnp.float32),
    )(tab, w)


def _sc_node_gather(t, idx):
    @functools.partial(
        pl.kernel,
        compiler_params=_sc_compiler_params(),
        out_type=jax.ShapeDtypeStruct((N, D), jnp.float32),
        mesh=_sc_mesh(),
        scratch_types=[
            pltpu.VMEM((CH,), jnp.int32),
            pltpu.VMEM((CH, D), jnp.float32),
            pltpu.VMEM((G_TAIL,), jnp.int32),
            pltpu.VMEM((G_TAIL, D), jnp.float32),
            pltpu.SemaphoreType.DMA,
        ],
    )
    def k(t_hbm, idx_hbm, y_hbm, cidx_v, rows_v, cidx_t, rows_t, sem):
        c = lax.axis_index("c")
        s = lax.axis_index("s")
        gwid = c * NS + s

        def chunk(base):
            pltpu.sync_copy(idx_hbm.at[pl.ds(base, CH)], cidx_v)
            pltpu.async_copy(t_hbm.at[cidx_v], rows_v, sem).wait()
            pltpu.sync_copy(rows_v, y_hbm.at[pl.ds(base, CH)])

        @pl.loop(0, G_ROUNDS)
        def _(j):
            chunk(pl.multiple_of((j * NW + gwid) * CH, 8))

        @pl.when(gwid < G_EXTRA)
        def _():
            chunk(pl.multiple_of((G_ROUNDS * NW + gwid) * CH, 8))

        @pl.when(gwid == G_EXTRA)
        def _():
            base = pl.multiple_of(G_TAIL_BASE, 8)
            pltpu.sync_copy(idx_hbm.at[pl.ds(base, G_TAIL)], cidx_t)
            pltpu.async_copy(t_hbm.at[cidx_t], rows_t, sem).wait()
            pltpu.sync_copy(rows_t, y_hbm.at[pl.ds(base, G_TAIL)])

    return k(t, idx)


def _sc_edge_pass(y, src, dst):
    @functools.partial(
        pl.kernel,
        compiler_params=_sc_compiler_params(),
        out_type=(
            jax.ShapeDtypeStruct((NC * N, D), jnp.float32),
            jax.ShapeDtypeStruct((NW * 16,), jnp.float32),
        ),
        mesh=_sc_mesh(),
        scratch_types=[
            pltpu.VMEM_SHARED((N, D), jnp.float32),
            pltpu.VMEM((CH,), jnp.int32),
            pltpu.VMEM((CH,), jnp.int32),
            pltpu.VMEM((CH,), jnp.int32),
            pltpu.VMEM((CH,), jnp.int32),
            pltpu.VMEM((CH, D), jnp.float32),
            pltpu.VMEM((CH, D), jnp.float32),
            pltpu.VMEM((CH, D), jnp.float32),
            pltpu.VMEM((CH, D), jnp.float32),
            pltpu.VMEM((CH, D), jnp.float32),
            pltpu.VMEM((ZR, D), jnp.float32),
            pltpu.VMEM((16,), jnp.float32),
            pltpu.SemaphoreType.DMA,
            pltpu.SemaphoreType.DMA,
        ],
    )
    def k(y_hbm, src_hbm, dst_hbm, acc_hbm, z_hbm,
          acc_sh, src_a, dst_a, src_b, dst_b, srow_a, drow_a, srow_b, drow_b,
          msg_v, zero_v, z_v, sem_a, sem_b):
        c = lax.axis_index("c")
        s = lax.axis_index("s")
        gwid = c * NS + s

        zeros16 = jnp.zeros((16,), jnp.float32)

        @pl.loop(0, ZR)
        def _(i):
            zero_v[i, pl.ds(0, 16)] = zeros16
            zero_v[i, pl.ds(16, 16)] = zeros16

        z_v[...] = zeros16

        def zero_rows(start_row, nrows):
            for off in range(0, nrows, ZR):
                sz = min(ZR, nrows - off)
                pltpu.sync_copy(zero_v.at[pl.ds(0, sz)],
                                acc_sh.at[pl.ds(start_row + off, sz)])

        @pl.when(s < BIG_TILES)
        def _():
            zero_rows(pl.multiple_of(s * ROWS_BIG, 8), ROWS_BIG)

        @pl.when(s >= BIG_TILES)
        def _():
            zero_rows(pl.multiple_of(
                BIG_TILES * ROWS_BIG + (s - BIG_TILES) * ROWS_SMALL, 8),
                ROWS_SMALL)

        plsc.subcore_barrier()

        lane = lax.iota(jnp.int32, 16)

        def ids_load(sv, dv, base):
            pltpu.sync_copy(src_hbm.at[pl.ds(base, CH)], sv)
            pltpu.sync_copy(dst_hbm.at[pl.ds(base, CH)], dv)

        def rows_start(sv, dv, sr, dr, sem):
            pltpu.async_copy(y_hbm.at[sv], sr, sem)
            pltpu.async_copy(y_hbm.at[dv], dr, sem)

        def rows_wait(sv, dv, sr, dr, sem):
            pltpu.make_async_copy(y_hbm.at[sv], sr, sem).wait()
            pltpu.make_async_copy(y_hbm.at[dv], dr, sem).wait()

        def compute_scatter(sr, dr, dv):
            @pl.loop(0, CH // 16)
            def _(g):
                base = g * 16
                w = jnp.zeros((16,), jnp.float32)
                for l in range(16):
                    a0 = sr[base + l, pl.ds(0, 16)]
                    a1 = sr[base + l, pl.ds(16, 16)]
                    b0 = dr[base + l, pl.ds(0, 16)]
                    b1 = dr[base + l, pl.ds(16, 16)]
                    s = jnp.sum(a0 * b0 + a1 * b1)
                    w = jnp.where(lane == l, s, w)
                w = jnp.exp(jnp.maximum(w, 0.2 * w))
                z_v[...] = z_v[...] + w
                for l in range(16):
                    wb = w[l]
                    a0 = sr[base + l, pl.ds(0, 16)]
                    a1 = sr[base + l, pl.ds(16, 16)]
                    msg_v[base + l, pl.ds(0, 16)] = wb * a0
                    msg_v[base + l, pl.ds(16, 16)] = wb * a1

            pltpu.sync_copy(msg_v, acc_sh.at[dv], add=True)

        tile_base = gwid * EPT

        ids_load(src_a, dst_a, pl.multiple_of(tile_base, 8))
        rows_start(src_a, dst_a, srow_a, drow_a, sem_a)

        @pl.loop(0, (NFULL - 1) // 2)
        def _(p):
            b1 = pl.multiple_of(tile_base + (2 * p + 1) * CH, 8)
            ids_load(src_b, dst_b, b1)
            rows_start(src_b, dst_b, srow_b, drow_b, sem_b)
            rows_wait(src_a, dst_a, srow_a, drow_a, sem_a)
            compute_scatter(srow_a, drow_a, dst_a)
            b2 = pl.multiple_of(tile_base + (2 * p + 2) * CH, 8)
            ids_load(src_a, dst_a, b2)
            rows_start(src_a, dst_a, srow_a, drow_a, sem_a)
            rows_wait(src_b, dst_b, srow_b, drow_b, sem_b)
            compute_scatter(srow_b, drow_b, dst_b)

        rows_wait(src_a, dst_a, srow_a, drow_a, sem_a)
        compute_scatter(srow_a, drow_a, dst_a)

        @pl.when(gwid < REM_CHUNKS)
        def _():
            base = pl.multiple_of(REM_BASE + gwid * CH, 8)
            ids_load(src_b, dst_b, base)
            rows_start(src_b, dst_b, srow_b, drow_b, sem_b)
            rows_wait(src_b, dst_b, srow_b, drow_b, sem_b)
            compute_scatter(srow_b, drow_b, dst_b)

        plsc.subcore_barrier()

        @pl.when(s < BIG_TILES)
        def _():
            rs = pl.multiple_of(s * ROWS_BIG, 8)
            pltpu.sync_copy(acc_sh.at[pl.ds(rs, ROWS_BIG)],
                            acc_hbm.at[pl.ds(c * N + rs, ROWS_BIG)])

        @pl.when(s >= BIG_TILES)
        def _():
            rs = pl.multiple_of(
                BIG_TILES * ROWS_BIG + (s - BIG_TILES) * ROWS_SMALL, 8)
            pltpu.sync_copy(acc_sh.at[pl.ds(rs, ROWS_SMALL)],
                            acc_hbm.at[pl.ds(c * N + rs, ROWS_SMALL)])

        pltpu.sync_copy(z_v, z_hbm.at[pl.ds(pl.multiple_of(gwid * 16, 8), 16)])

    return k(y, src, dst)


def _combine_body(a0_ref, a1_ref, z_ref, o_ref):
    zsum = jnp.sum(z_ref[...])
    o_ref[...] = jnp.maximum((a0_ref[...] + a1_ref[...]) / zsum, 0.0)


def _combine(acc, z):
    nblk = N // MM_BLK
    return pl.pallas_call(
        _combine_body,
        grid=(nblk,),
        in_specs=[
            pl.BlockSpec((MM_BLK, D), lambda i: (i, 0)),
            pl.BlockSpec((MM_BLK, D), lambda i, _n=nblk: (i + _n, 0)),
            pl.BlockSpec((NW, 16), lambda i: (0, 0)),
        ],
        out_specs=pl.BlockSpec((MM_BLK, D), lambda i: (i, 0)),
        out_shape=jax.ShapeDtypeStruct((N, D), jnp.float32),
    )(acc, acc, z)


def kernel(user_emb, entity_emb, W, W_r, user_indices, item_indices,
           edge_index_ui, edge_index_kg, edge_type_kg):
    tab = jnp.concatenate([user_emb, entity_emb], axis=0)
    t = _transform(tab, W)
    idx = jnp.concatenate([user_indices.astype(jnp.int32),
                           item_indices.astype(jnp.int32) + N_U])
    src = edge_index_ui[0].astype(jnp.int32)
    dst = edge_index_ui[1].astype(jnp.int32)
    y = _sc_node_gather(t, idx)
    acc, z = _sc_edge_pass(y, src, dst)
    x = _combine(acc, z.reshape(NW, 16))
    return (x[:N_U], x[N_U:])


# speedup vs baseline: 13.5065x; 1.3433x over previous
import dataclasses
import functools

import jax
import jax.numpy as jnp
from jax import lax
from jax.experimental import pallas as pl
from jax.experimental.pallas import tpu as pltpu
from jax.experimental.pallas import tpu_sc as plsc

N_U = 25000
N_E = 25000
N = N_U + N_E
D = 32
E = 800000
NC, NS = 2, 16
NW = NC * NS
CH = 128
NFULL = 195
EPT = NFULL * CH
REM_BASE = EPT * NW
REM_CHUNKS = (E - REM_BASE) // CH
NCHT = EPT // CH
NCH_PAD = 6256

GFULL = 390
G_ROUNDS = GFULL // NW
G_EXTRA = GFULL - G_ROUNDS * NW
G_TAIL_BASE = GFULL * CH
G_TAIL = N - G_TAIL_BASE
ROWS_BIG = 3128
ROWS_SMALL = 3120
BIG_TILES = 10
ZR = 64
MM_BLK = 2000


def _sc_compiler_params():
    cp = pltpu.CompilerParams()
    fields = pltpu.CompilerParams.__dataclass_fields__
    if "needs_layout_passes" in fields:
        cp = dataclasses.replace(cp, needs_layout_passes=False)
    if "use_tc_tiling_on_sc" in fields:
        cp = dataclasses.replace(cp, use_tc_tiling_on_sc=False)
    return cp


def _sc_mesh():
    return plsc.VectorSubcoreMesh(core_axis_name="c", subcore_axis_name="s",
                                  num_cores=NC, num_subcores=NS)


def _xw_body(x_ref, w_ref, o_ref):
    o_ref[...] = jnp.dot(x_ref[...], w_ref[...],
                         preferred_element_type=jnp.float32)


def _transform(tab, w):
    return pl.pallas_call(
        _xw_body,
        grid=(N // MM_BLK,),
        in_specs=[
            pl.BlockSpec((MM_BLK, D), lambda i: (i, 0)),
            pl.BlockSpec((D, D), lambda i: (0, 0)),
        ],
        out_specs=pl.BlockSpec((MM_BLK, D), lambda i: (i, 0)),
        out_shape=jax.ShapeDtypeStruct((N, D), jnp.float32),
    )(tab, w)


def _sc_node_gather(t, idx):
    @functools.partial(
        pl.kernel,
        compiler_params=_sc_compiler_params(),
        out_type=jax.ShapeDtypeStruct((N, D), jnp.float32),
        mesh=_sc_mesh(),
        scratch_types=[
            pltpu.VMEM((CH,), jnp.int32),
            pltpu.VMEM((CH, D), jnp.float32),
            pltpu.VMEM((G_TAIL,), jnp.int32),
            pltpu.VMEM((G_TAIL, D), jnp.float32),
            pltpu.SemaphoreType.DMA,
        ],
    )
    def k(t_hbm, idx_hbm, y_hbm, cidx_v, rows_v, cidx_t, rows_t, sem):
        c = lax.axis_index("c")
        s = lax.axis_index("s")
        gwid = c * NS + s

        def chunk(base):
            pltpu.sync_copy(idx_hbm.at[pl.ds(base, CH)], cidx_v)
            pltpu.async_copy(t_hbm.at[cidx_v], rows_v, sem).wait()
            pltpu.sync_copy(rows_v, y_hbm.at[pl.ds(base, CH)])

        @pl.loop(0, G_ROUNDS)
        def _(j):
            chunk(pl.multiple_of((j * NW + gwid) * CH, 8))

        @pl.when(gwid < G_EXTRA)
        def _():
            chunk(pl.multiple_of((G_ROUNDS * NW + gwid) * CH, 8))

        @pl.when(gwid == G_EXTRA)
        def _():
            base = pl.multiple_of(G_TAIL_BASE, 8)
            pltpu.sync_copy(idx_hbm.at[pl.ds(base, G_TAIL)], cidx_t)
            pltpu.async_copy(t_hbm.at[cidx_t], rows_t, sem).wait()
            pltpu.sync_copy(rows_t, y_hbm.at[pl.ds(base, G_TAIL)])

    return k(t, idx)


def _sc_edge_pass(y, src, dst):
    @functools.partial(
        pl.kernel,
        compiler_params=_sc_compiler_params(),
        out_type=(
            jax.ShapeDtypeStruct((NC * N, D), jnp.float32),
            jax.ShapeDtypeStruct((NW * 16,), jnp.float32),
        ),
        mesh=_sc_mesh(),
        scratch_types=[
            pltpu.VMEM_SHARED((N, D), jnp.float32),
            pltpu.VMEM((32, CH), jnp.int32),
            pltpu.VMEM((32, CH), jnp.int32),
            pltpu.VMEM((CH, D), jnp.float32),
            pltpu.VMEM((CH, D), jnp.float32),
            pltpu.VMEM((CH, D), jnp.float32),
            pltpu.VMEM((CH, D), jnp.float32),
            pltpu.VMEM((CH, D), jnp.float32),
            pltpu.VMEM((ZR, D), jnp.float32),
            pltpu.VMEM((16,), jnp.float32),
            pltpu.SemaphoreType.DMA,
            pltpu.SemaphoreType.DMA,
        ],
    )
    def k(y_hbm, src_hbm, dst_hbm, acc_hbm, z_hbm,
          acc_sh, sbig, dbig, srow_a, drow_a, srow_b, drow_b,
          msg_v, zero_v, z_v, sem_a, sem_b):
        c = lax.axis_index("c")
        s = lax.axis_index("s")
        gwid = c * NS + s

        zeros16 = jnp.zeros((16,), jnp.float32)

        @pl.loop(0, ZR)
        def _(i):
            zero_v[i, pl.ds(0, 16)] = zeros16
            zero_v[i, pl.ds(16, 16)] = zeros16

        z_v[...] = zeros16

        def zero_rows(start_row, nrows):
            for off in range(0, nrows, ZR):
                sz = min(ZR, nrows - off)
                pltpu.sync_copy(zero_v.at[pl.ds(0, sz)],
                                acc_sh.at[pl.ds(start_row + off, sz)])

        @pl.when(s < BIG_TILES)
        def _():
            zero_rows(pl.multiple_of(s * ROWS_BIG, 8), ROWS_BIG)

        @pl.when(s >= BIG_TILES)
        def _():
            zero_rows(pl.multiple_of(
                BIG_TILES * ROWS_BIG + (s - BIG_TILES) * ROWS_SMALL, 8),
                ROWS_SMALL)

        plsc.subcore_barrier()

        lane = lax.iota(jnp.int32, 16)

        def batch_load(b):
            half = lax.rem(b, 2) * 16
            c0 = gwid * NCHT + b * 16
            pltpu.sync_copy(src_hbm.at[pl.ds(c0, 16)],
                            sbig.at[pl.ds(half, 16)])
            pltpu.sync_copy(dst_hbm.at[pl.ds(c0, 16)],
                            dbig.at[pl.ds(half, 16)])

        def rows_start(row, sr, dr, sem):
            pltpu.async_copy(y_hbm.at[sbig.at[row]], sr, sem)
            pltpu.async_copy(y_hbm.at[dbig.at[row]], dr, sem)

        def rows_wait(row, sr, dr, sem):
            pltpu.make_async_copy(y_hbm.at[sbig.at[row]], sr, sem).wait()
            pltpu.make_async_copy(y_hbm.at[dbig.at[row]], dr, sem).wait()

        def compute_scatter(sr, dr, row):
            @pl.loop(0, CH // 16)
            def _(g):
                base = g * 16
                w = jnp.zeros((16,), jnp.float32)
                for l in range(16):
                    a0 = sr[base + l, pl.ds(0, 16)]
                    a1 = sr[base + l, pl.ds(16, 16)]
                    b0 = dr[base + l, pl.ds(0, 16)]
                    b1 = dr[base + l, pl.ds(16, 16)]
                    s = jnp.sum(a0 * b0 + a1 * b1)
                    w = jnp.where(lane == l, s, w)
                w = jnp.exp(jnp.maximum(w, 0.2 * w))
                z_v[...] = z_v[...] + w
                for l in range(16):
                    wb = w[l]
                    a0 = sr[base + l, pl.ds(0, 16)]
                    a1 = sr[base + l, pl.ds(16, 16)]
                    msg_v[base + l, pl.ds(0, 16)] = wb * a0
                    msg_v[base + l, pl.ds(16, 16)] = wb * a1

            pltpu.sync_copy(msg_v, acc_sh.at[dbig.at[row]], add=True)

        batch_load(0)
        rows_start(0, srow_a, drow_a, sem_a)

        @pl.loop(0, (NFULL - 1) // 2)
        def _(p):
            @pl.when(lax.rem(p, 8) == 4)
            def _():
                batch_load((p + 4) // 8)

            j_a = 2 * p
            j_b = j_a + 1
            r_b = lax.rem(j_b, 32)
            rows_start(r_b, srow_b, drow_b, sem_b)
            r_a = lax.rem(j_a, 32)
            rows_wait(r_a, srow_a, drow_a, sem_a)
            compute_scatter(srow_a, drow_a, r_a)
            r_a2 = lax.rem(j_a + 2, 32)
            rows_start(r_a2, srow_a, drow_a, sem_a)
            rows_wait(r_b, srow_b, drow_b, sem_b)
            compute_scatter(srow_b, drow_b, r_b)

        r_last = (NFULL - 1) % 32
        rows_wait(r_last, srow_a, drow_a, sem_a)
        compute_scatter(srow_a, drow_a, r_last)

        @pl.when(gwid < REM_CHUNKS)
        def _():
            rc = REM_BASE // CH + gwid
            pltpu.sync_copy(src_hbm.at[pl.ds(rc, 1)], sbig.at[pl.ds(0, 1)])
            pltpu.sync_copy(dst_hbm.at[pl.ds(rc, 1)], dbig.at[pl.ds(0, 1)])
            rows_start(0, srow_b, drow_b, sem_b)
            rows_wait(0, srow_b, drow_b, sem_b)
            compute_scatter(srow_b, drow_b, 0)

        plsc.subcore_barrier()

        @pl.when(s < BIG_TILES)
        def _():
            rs = pl.multiple_of(s * ROWS_BIG, 8)
            pltpu.sync_copy(acc_sh.at[pl.ds(rs, ROWS_BIG)],
                            acc_hbm.at[pl.ds(c * N + rs, ROWS_BIG)])

        @pl.when(s >= BIG_TILES)
        def _():
            rs = pl.multiple_of(
                BIG_TILES * ROWS_BIG + (s - BIG_TILES) * ROWS_SMALL, 8)
            pltpu.sync_copy(acc_sh.at[pl.ds(rs, ROWS_SMALL)],
                            acc_hbm.at[pl.ds(c * N + rs, ROWS_SMALL)])

        pltpu.sync_copy(z_v, z_hbm.at[pl.ds(pl.multiple_of(gwid * 16, 8), 16)])

    return k(y, src, dst)


def _combine_body(a0_ref, a1_ref, z_ref, o_ref):
    zsum = jnp.sum(z_ref[...])
    o_ref[...] = jnp.maximum((a0_ref[...] + a1_ref[...]) / zsum, 0.0)


def _combine(acc, z):
    nblk = N // MM_BLK
    return pl.pallas_call(
        _combine_body,
        grid=(nblk,),
        in_specs=[
            pl.BlockSpec((MM_BLK, D), lambda i: (i, 0)),
            pl.BlockSpec((MM_BLK, D), lambda i, _n=nblk: (i + _n, 0)),
            pl.BlockSpec((NW, 16), lambda i: (0, 0)),
        ],
        out_specs=pl.BlockSpec((MM_BLK, D), lambda i: (i, 0)),
        out_shape=jax.ShapeDtypeStruct((N, D), jnp.float32),
    )(acc, acc, z)


def kernel(user_emb, entity_emb, W, W_r, user_indices, item_indices,
           edge_index_ui, edge_index_kg, edge_type_kg):
    tab = jnp.concatenate([user_emb, entity_emb], axis=0)
    t = _transform(tab, W)
    idx = jnp.concatenate([user_indices.astype(jnp.int32),
                           item_indices.astype(jnp.int32) + N_U])
    pad = jnp.zeros((NCH_PAD * CH - E,), jnp.int32)
    src = jnp.concatenate([edge_index_ui[0].astype(jnp.int32), pad])
    dst = jnp.concatenate([edge_index_ui[1].astype(jnp.int32), pad])
    y = _sc_node_gather(t, idx)
    acc, z = _sc_edge_pass(y, src.reshape(NCH_PAD, CH),
                           dst.reshape(NCH_PAD, CH))
    x = _combine(acc, z.reshape(NW, 16))
    return (x[:N_U], x[N_U:])


# speedup vs baseline: 13.7181x; 1.0157x over previous
import dataclasses
import functools

import jax
import jax.numpy as jnp
from jax import lax
from jax.experimental import pallas as pl
from jax.experimental.pallas import tpu as pltpu
from jax.experimental.pallas import tpu_sc as plsc

N_U = 25000
N_E = 25000
N = N_U + N_E
D = 32
E = 800000
NC, NS = 2, 16
NW = NC * NS
CH = 128
NFULL = 195
EPT = NFULL * CH
REM_BASE = EPT * NW
REM_CHUNKS = (E - REM_BASE) // CH
NCHT = EPT // CH
NCH_PAD = 6256

GFULL = 390
G_ROUNDS = GFULL // NW
G_EXTRA = GFULL - G_ROUNDS * NW
G_TAIL_BASE = GFULL * CH
G_TAIL = N - G_TAIL_BASE
ROWS_BIG = 3128
ROWS_SMALL = 3120
BIG_TILES = 10
ZR = 64
MM_BLK = 2000


def _sc_compiler_params():
    cp = pltpu.CompilerParams()
    fields = pltpu.CompilerParams.__dataclass_fields__
    if "needs_layout_passes" in fields:
        cp = dataclasses.replace(cp, needs_layout_passes=False)
    if "use_tc_tiling_on_sc" in fields:
        cp = dataclasses.replace(cp, use_tc_tiling_on_sc=False)
    return cp


def _sc_mesh():
    return plsc.VectorSubcoreMesh(core_axis_name="c", subcore_axis_name="s",
                                  num_cores=NC, num_subcores=NS)


def _xw_body(x_ref, w_ref, o_ref):
    o_ref[...] = jnp.dot(x_ref[...], w_ref[...],
                         preferred_element_type=jnp.float32)


def _transform(tab, w):
    return pl.pallas_call(
        _xw_body,
        grid=(N // MM_BLK,),
        in_specs=[
            pl.BlockSpec((MM_BLK, D), lambda i: (i, 0)),
            pl.BlockSpec((D, D), lambda i: (0, 0)),
        ],
        out_specs=pl.BlockSpec((MM_BLK, D), lambda i: (i, 0)),
        out_shape=jax.ShapeDtypeStruct((N, D), jnp.float32),
    )(tab, w)


def _sc_node_gather(t, idx):
    @functools.partial(
        pl.kernel,
        compiler_params=_sc_compiler_params(),
        out_type=jax.ShapeDtypeStruct((N, D), jnp.float32),
        mesh=_sc_mesh(),
        scratch_types=[
            pltpu.VMEM((CH,), jnp.int32),
            pltpu.VMEM((CH, D), jnp.float32),
            pltpu.VMEM((G_TAIL,), jnp.int32),
            pltpu.VMEM((G_TAIL, D), jnp.float32),
            pltpu.SemaphoreType.DMA,
        ],
    )
    def k(t_hbm, idx_hbm, y_hbm, cidx_v, rows_v, cidx_t, rows_t, sem):
        c = lax.axis_index("c")
        s = lax.axis_index("s")
        gwid = c * NS + s

        def chunk(base):
            pltpu.sync_copy(idx_hbm.at[pl.ds(base, CH)], cidx_v)
            pltpu.async_copy(t_hbm.at[cidx_v], rows_v, sem).wait()
            pltpu.sync_copy(rows_v, y_hbm.at[pl.ds(base, CH)])

        @pl.loop(0, G_ROUNDS)
        def _(j):
            chunk(pl.multiple_of((j * NW + gwid) * CH, 8))

        @pl.when(gwid < G_EXTRA)
        def _():
            chunk(pl.multiple_of((G_ROUNDS * NW + gwid) * CH, 8))

        @pl.when(gwid == G_EXTRA)
        def _():
            base = pl.multiple_of(G_TAIL_BASE, 8)
            pltpu.sync_copy(idx_hbm.at[pl.ds(base, G_TAIL)], cidx_t)
            pltpu.async_copy(t_hbm.at[cidx_t], rows_t, sem).wait()
            pltpu.sync_copy(rows_t, y_hbm.at[pl.ds(base, G_TAIL)])

    return k(t, idx)


def _sc_edge_pass(y, src, dst):
    @functools.partial(
        pl.kernel,
        compiler_params=_sc_compiler_params(),
        out_type=(
            jax.ShapeDtypeStruct((NC * N, D), jnp.float32),
            jax.ShapeDtypeStruct((NW * 16,), jnp.float32),
        ),
        mesh=_sc_mesh(),
        scratch_types=[
            pltpu.VMEM_SHARED((N, D), jnp.float32),
            pltpu.VMEM((16, CH), jnp.int32),
            pltpu.VMEM((16, CH), jnp.int32),
            pltpu.VMEM((CH, D), jnp.float32),
            pltpu.VMEM((CH, D), jnp.float32),
            pltpu.VMEM((CH, D), jnp.float32),
            pltpu.VMEM((CH, D), jnp.float32),
            pltpu.VMEM((CH, D), jnp.float32),
            pltpu.VMEM((CH, D), jnp.float32),
            pltpu.VMEM((ZR, D), jnp.float32),
            pltpu.VMEM((16,), jnp.float32),
            pltpu.SemaphoreType.DMA,
            pltpu.SemaphoreType.DMA,
            pltpu.SemaphoreType.DMA,
            pltpu.SemaphoreType.DMA,
        ],
    )
    def k(y_hbm, src_hbm, dst_hbm, acc_hbm, z_hbm,
          acc_sh, sbig, dbig, srow_a, drow_a, srow_b, drow_b,
          msg_a, msg_b, zero_v, z_v, sem_a, sem_b, sem_sa, sem_sb):
        c = lax.axis_index("c")
        s = lax.axis_index("s")
        gwid = c * NS + s

        zeros16 = jnp.zeros((16,), jnp.float32)

        @pl.loop(0, ZR)
        def _(i):
            zero_v[i, pl.ds(0, 16)] = zeros16
            zero_v[i, pl.ds(16, 16)] = zeros16

        z_v[...] = zeros16

        def zero_rows(start_row, nrows):
            for off in range(0, nrows, ZR):
                sz = min(ZR, nrows - off)
                pltpu.sync_copy(zero_v.at[pl.ds(0, sz)],
                                acc_sh.at[pl.ds(start_row + off, sz)])

        @pl.when(s < BIG_TILES)
        def _():
            zero_rows(pl.multiple_of(s * ROWS_BIG, 8), ROWS_BIG)

        @pl.when(s >= BIG_TILES)
        def _():
            zero_rows(pl.multiple_of(
                BIG_TILES * ROWS_BIG + (s - BIG_TILES) * ROWS_SMALL, 8),
                ROWS_SMALL)

        plsc.subcore_barrier()

        lane = lax.iota(jnp.int32, 16)

        def batch_load(b):
            half = lax.rem(b, 2) * 8
            c0 = gwid * NCHT + b * 8
            pltpu.sync_copy(src_hbm.at[pl.ds(c0, 8)],
                            sbig.at[pl.ds(half, 8)])
            pltpu.sync_copy(dst_hbm.at[pl.ds(c0, 8)],
                            dbig.at[pl.ds(half, 8)])

        def rows_start(row, sr, dr, sem):
            pltpu.async_copy(y_hbm.at[sbig.at[row]], sr, sem)
            pltpu.async_copy(y_hbm.at[dbig.at[row]], dr, sem)

        def rows_wait(row, sr, dr, sem):
            pltpu.make_async_copy(y_hbm.at[sbig.at[row]], sr, sem).wait()
            pltpu.make_async_copy(y_hbm.at[dbig.at[row]], dr, sem).wait()

        def scat_start(mv, row, sem_s):
            pltpu.async_copy(mv, acc_sh.at[dbig.at[row]], sem_s, add=True)

        def scat_wait(mv, sem_s):
            pltpu.make_async_copy(mv, acc_sh.at[dbig.at[0]], sem_s).wait()

        def compute(sr, dr, mv):
            @pl.loop(0, CH // 16)
            def _(g):
                base = g * 16
                w = jnp.zeros((16,), jnp.float32)
                for l in range(16):
                    a0 = sr[base + l, pl.ds(0, 16)]
                    a1 = sr[base + l, pl.ds(16, 16)]
                    b0 = dr[base + l, pl.ds(0, 16)]
                    b1 = dr[base + l, pl.ds(16, 16)]
                    s = jnp.sum(a0 * b0 + a1 * b1)
                    w = jnp.where(lane == l, s, w)
                w = jnp.exp(jnp.maximum(w, 0.2 * w))
                z_v[...] = z_v[...] + w
                for l in range(16):
                    wb = w[l]
                    a0 = sr[base + l, pl.ds(0, 16)]
                    a1 = sr[base + l, pl.ds(16, 16)]
                    mv[base + l, pl.ds(0, 16)] = wb * a0
                    mv[base + l, pl.ds(16, 16)] = wb * a1

        batch_load(0)
        rows_start(0, srow_a, drow_a, sem_a)

        @pl.loop(0, (NFULL - 1) // 2)
        def _(p):
            @pl.when(lax.rem(p, 4) == 2)
            def _():
                batch_load((p + 2) // 4)

            j_a = 2 * p
            j_b = j_a + 1
            r_b = lax.rem(j_b, 16)
            rows_start(r_b, srow_b, drow_b, sem_b)
            r_a = lax.rem(j_a, 16)
            rows_wait(r_a, srow_a, drow_a, sem_a)

            @pl.when(p > 0)
            def _():
                scat_wait(msg_a, sem_sa)

            compute(srow_a, drow_a, msg_a)
            scat_start(msg_a, r_a, sem_sa)
            r_a2 = lax.rem(j_a + 2, 16)
            rows_start(r_a2, srow_a, drow_a, sem_a)
            rows_wait(r_b, srow_b, drow_b, sem_b)

            @pl.when(p > 0)
            def _():
                scat_wait(msg_b, sem_sb)

            compute(srow_b, drow_b, msg_b)
            scat_start(msg_b, r_b, sem_sb)

        scat_wait(msg_a, sem_sa)
        r_last = (NFULL - 1) % 16
        rows_wait(r_last, srow_a, drow_a, sem_a)
        compute(srow_a, drow_a, msg_a)
        scat_start(msg_a, r_last, sem_sa)
        scat_wait(msg_a, sem_sa)
        scat_wait(msg_b, sem_sb)

        @pl.when(gwid < REM_CHUNKS)
        def _():
            rc = REM_BASE // CH + gwid
            pltpu.sync_copy(src_hbm.at[pl.ds(rc, 1)], sbig.at[pl.ds(0, 1)])
            pltpu.sync_copy(dst_hbm.at[pl.ds(rc, 1)], dbig.at[pl.ds(0, 1)])
            rows_start(0, srow_b, drow_b, sem_b)
            rows_wait(0, srow_b, drow_b, sem_b)
            compute(srow_b, drow_b, msg_b)
            pltpu.sync_copy(msg_b, acc_sh.at[dbig.at[0]], add=True)

        plsc.subcore_barrier()

        @pl.when(s < BIG_TILES)
        def _():
            rs = pl.multiple_of(s * ROWS_BIG, 8)
            pltpu.sync_copy(acc_sh.at[pl.ds(rs, ROWS_BIG)],
                            acc_hbm.at[pl.ds(c * N + rs, ROWS_BIG)])

        @pl.when(s >= BIG_TILES)
        def _():
            rs = pl.multiple_of(
                BIG_TILES * ROWS_BIG + (s - BIG_TILES) * ROWS_SMALL, 8)
            pltpu.sync_copy(acc_sh.at[pl.ds(rs, ROWS_SMALL)],
                            acc_hbm.at[pl.ds(c * N + rs, ROWS_SMALL)])

        pltpu.sync_copy(z_v, z_hbm.at[pl.ds(pl.multiple_of(gwid * 16, 8), 16)])

    return k(y, src, dst)


def _combine_body(a0_ref, a1_ref, z_ref, o_ref):
    zsum = jnp.sum(z_ref[...])
    o_ref[...] = jnp.maximum((a0_ref[...] + a1_ref[...]) / zsum, 0.0)


def _combine(acc, z):
    nblk = N // MM_BLK
    return pl.pallas_call(
        _combine_body,
        grid=(nblk,),
        in_specs=[
            pl.BlockSpec((MM_BLK, D), lambda i: (i, 0)),
            pl.BlockSpec((MM_BLK, D), lambda i, _n=nblk: (i + _n, 0)),
            pl.BlockSpec((NW, 16), lambda i: (0, 0)),
        ],
        out_specs=pl.BlockSpec((MM_BLK, D), lambda i: (i, 0)),
        out_shape=jax.ShapeDtypeStruct((N, D), jnp.float32),
    )(acc, acc, z)


def kernel(user_emb, entity_emb, W, W_r, user_indices, item_indices,
           edge_index_ui, edge_index_kg, edge_type_kg):
    tab = jnp.concatenate([user_emb, entity_emb], axis=0)
    t = _transform(tab, W)
    idx = jnp.concatenate([user_indices.astype(jnp.int32),
                           item_indices.astype(jnp.int32) + N_U])
    pad = jnp.zeros((NCH_PAD * CH - E,), jnp.int32)
    src = jnp.concatenate([edge_index_ui[0].astype(jnp.int32), pad])
    dst = jnp.concatenate([edge_index_ui[1].astype(jnp.int32), pad])
    y = _sc_node_gather(t, idx)
    acc, z = _sc_edge_pass(y, src.reshape(NCH_PAD, CH),
                           dst.reshape(NCH_PAD, CH))
    x = _combine(acc, z.reshape(NW, 16))
    return (x[:N_U], x[N_U:])


# speedup vs baseline: 14.7147x; 1.0727x over previous
import dataclasses
import functools

import jax
import jax.numpy as jnp
from jax import lax
from jax.experimental import pallas as pl
from jax.experimental.pallas import tpu as pltpu
from jax.experimental.pallas import tpu_sc as plsc

N_U = 25000
N_E = 25000
N = N_U + N_E
D = 32
E = 800000
NC, NS = 2, 16
NW = NC * NS
CH = 128
NFULL = 195
EPT = NFULL * CH
REM_BASE = EPT * NW
REM_CHUNKS = (E - REM_BASE) // CH
NCHT = EPT // CH
NCH_PAD = 6256

GFULL = 390
G_ROUNDS = GFULL // NW
G_EXTRA = GFULL - G_ROUNDS * NW
G_TAIL_BASE = GFULL * CH
G_TAIL = N - G_TAIL_BASE
ROWS_BIG = 3128
ROWS_SMALL = 3120
BIG_TILES = 10
ZR = 64
MM_BLK = 2000


def _sc_compiler_params():
    cp = pltpu.CompilerParams()
    fields = pltpu.CompilerParams.__dataclass_fields__
    if "needs_layout_passes" in fields:
        cp = dataclasses.replace(cp, needs_layout_passes=False)
    if "use_tc_tiling_on_sc" in fields:
        cp = dataclasses.replace(cp, use_tc_tiling_on_sc=False)
    return cp


def _sc_mesh():
    return plsc.VectorSubcoreMesh(core_axis_name="c", subcore_axis_name="s",
                                  num_cores=NC, num_subcores=NS)


def _xw_body(x_ref, w_ref, o_ref):
    o_ref[...] = jnp.dot(x_ref[...], w_ref[...],
                         preferred_element_type=jnp.float32)


def _transform(tab, w):
    return pl.pallas_call(
        _xw_body,
        grid=(N // MM_BLK,),
        in_specs=[
            pl.BlockSpec((MM_BLK, D), lambda i: (i, 0)),
            pl.BlockSpec((D, D), lambda i: (0, 0)),
        ],
        out_specs=pl.BlockSpec((MM_BLK, D), lambda i: (i, 0)),
        out_shape=jax.ShapeDtypeStruct((N, D), jnp.float32),
    )(tab, w)


def _sc_node_gather(t, idx):
    @functools.partial(
        pl.kernel,
        compiler_params=_sc_compiler_params(),
        out_type=jax.ShapeDtypeStruct((N, D), jnp.float32),
        mesh=_sc_mesh(),
        scratch_types=[
            pltpu.VMEM((CH,), jnp.int32),
            pltpu.VMEM((CH, D), jnp.float32),
            pltpu.VMEM((G_TAIL,), jnp.int32),
            pltpu.VMEM((G_TAIL, D), jnp.float32),
            pltpu.SemaphoreType.DMA,
        ],
    )
    def k(t_hbm, idx_hbm, y_hbm, cidx_v, rows_v, cidx_t, rows_t, sem):
        c = lax.axis_index("c")
        s = lax.axis_index("s")
        gwid = c * NS + s

        def chunk(base):
            pltpu.sync_copy(idx_hbm.at[pl.ds(base, CH)], cidx_v)
            pltpu.async_copy(t_hbm.at[cidx_v], rows_v, sem).wait()
            pltpu.sync_copy(rows_v, y_hbm.at[pl.ds(base, CH)])

        @pl.loop(0, G_ROUNDS)
        def _(j):
            chunk(pl.multiple_of((j * NW + gwid) * CH, 8))

        @pl.when(gwid < G_EXTRA)
        def _():
            chunk(pl.multiple_of((G_ROUNDS * NW + gwid) * CH, 8))

        @pl.when(gwid == G_EXTRA)
        def _():
            base = pl.multiple_of(G_TAIL_BASE, 8)
            pltpu.sync_copy(idx_hbm.at[pl.ds(base, G_TAIL)], cidx_t)
            pltpu.async_copy(t_hbm.at[cidx_t], rows_t, sem).wait()
            pltpu.sync_copy(rows_t, y_hbm.at[pl.ds(base, G_TAIL)])

    return k(t, idx)


def _sc_edge_pass(y, src, dst):
    @functools.partial(
        pl.kernel,
        compiler_params=_sc_compiler_params(),
        out_type=(
            jax.ShapeDtypeStruct((NC * N, D), jnp.float32),
            jax.ShapeDtypeStruct((NW * 16,), jnp.float32),
        ),
        mesh=_sc_mesh(),
        scratch_types=[
            pltpu.VMEM_SHARED((N, D), jnp.float32),
            pltpu.VMEM((16, CH), jnp.int32),
            pltpu.VMEM((16, CH), jnp.int32),
            pltpu.VMEM((CH, D), jnp.float32),
            pltpu.VMEM((CH, D), jnp.float32),
            pltpu.VMEM((CH, D), jnp.float32),
            pltpu.VMEM((CH, D), jnp.float32),
            pltpu.VMEM((CH, D), jnp.float32),
            pltpu.VMEM((CH, D), jnp.float32),
            pltpu.VMEM((ZR, D), jnp.float32),
            pltpu.VMEM((16,), jnp.float32),
            pltpu.SemaphoreType.DMA,
            pltpu.SemaphoreType.DMA,
            pltpu.SemaphoreType.DMA,
            pltpu.SemaphoreType.DMA,
            pltpu.SemaphoreType.DMA,
        ],
    )
    def k(y_hbm, src_hbm, dst_hbm, acc_hbm, z_hbm,
          acc_sh, sbig, dbig, srow_a, drow_a, srow_b, drow_b,
          msg_a, msg_b, zero_v, z_v, sem_a, sem_b, sem_sa, sem_sb, sem_i):
        c = lax.axis_index("c")
        s = lax.axis_index("s")
        gwid = c * NS + s

        zeros16 = jnp.zeros((16,), jnp.float32)

        @pl.loop(0, ZR)
        def _(i):
            zero_v[i, pl.ds(0, 16)] = zeros16
            zero_v[i, pl.ds(16, 16)] = zeros16

        z_v[...] = zeros16

        def zero_rows(start_row, nrows):
            for off in range(0, nrows, ZR):
                sz = min(ZR, nrows - off)
                pltpu.async_copy(zero_v.at[pl.ds(0, sz)],
                                 acc_sh.at[pl.ds(start_row + off, sz)],
                                 sem_i)
            for off in range(0, nrows, ZR):
                sz = min(ZR, nrows - off)
                pltpu.make_async_copy(
                    zero_v.at[pl.ds(0, sz)],
                    acc_sh.at[pl.ds(start_row + off, sz)], sem_i).wait()

        @pl.when(s < BIG_TILES)
        def _():
            zero_rows(pl.multiple_of(s * ROWS_BIG, 8), ROWS_BIG)

        @pl.when(s >= BIG_TILES)
        def _():
            zero_rows(pl.multiple_of(
                BIG_TILES * ROWS_BIG + (s - BIG_TILES) * ROWS_SMALL, 8),
                ROWS_SMALL)

        plsc.subcore_barrier()

        lane = lax.iota(jnp.int32, 16)

        def _batch_refs(b):
            half = lax.rem(b, 2) * 8
            c0 = gwid * NCHT + b * 8
            return ((src_hbm.at[pl.ds(c0, 8)], sbig.at[pl.ds(half, 8)]),
                    (dst_hbm.at[pl.ds(c0, 8)], dbig.at[pl.ds(half, 8)]))

        def batch_start(b):
            for s_, d_ in _batch_refs(b):
                pltpu.async_copy(s_, d_, sem_i)

        def batch_wait(b):
            for s_, d_ in _batch_refs(b):
                pltpu.make_async_copy(s_, d_, sem_i).wait()

        def batch_load(b):
            batch_start(b)
            batch_wait(b)

        def rows_start(row, sr, dr, sem):
            pltpu.async_copy(y_hbm.at[sbig.at[row]], sr, sem)
            pltpu.async_copy(y_hbm.at[dbig.at[row]], dr, sem)

        def rows_wait(row, sr, dr, sem):
            pltpu.make_async_copy(y_hbm.at[sbig.at[row]], sr, sem).wait()
            pltpu.make_async_copy(y_hbm.at[dbig.at[row]], dr, sem).wait()

        def scat_start(mv, row, sem_s):
            pltpu.async_copy(mv, acc_sh.at[dbig.at[row]], sem_s, add=True)

        def scat_wait(mv, sem_s):
            pltpu.make_async_copy(mv, acc_sh.at[dbig.at[0]], sem_s).wait()

        def compute(sr, dr, mv):
            @pl.loop(0, CH // 16)
            def _(g):
                base = g * 16
                w = jnp.zeros((16,), jnp.float32)
                for l in range(16):
                    a0 = sr[base + l, pl.ds(0, 16)]
                    a1 = sr[base + l, pl.ds(16, 16)]
                    b0 = dr[base + l, pl.ds(0, 16)]
                    b1 = dr[base + l, pl.ds(16, 16)]
                    s = jnp.sum(a0 * b0 + a1 * b1)
                    w = jnp.where(lane == l, s, w)
                w = jnp.exp(jnp.maximum(w, 0.2 * w))
                z_v[...] = z_v[...] + w
                for l in range(16):
                    wb = w[l]
                    a0 = sr[base + l, pl.ds(0, 16)]
                    a1 = sr[base + l, pl.ds(16, 16)]
                    mv[base + l, pl.ds(0, 16)] = wb * a0
                    mv[base + l, pl.ds(16, 16)] = wb * a1

        batch_load(0)
        rows_start(0, srow_a, drow_a, sem_a)

        @pl.loop(0, (NFULL - 1) // 2)
        def _(p):
            @pl.when(lax.rem(p, 4) == 1)
            def _():
                batch_start((p + 3) // 4)

            @pl.when(lax.rem(p, 4) == 2)
            def _():
                batch_wait((p + 2) // 4)

            j_a = 2 * p
            j_b = j_a + 1
            r_b = lax.rem(j_b, 16)
            rows_start(r_b, srow_b, drow_b, sem_b)
            r_a = lax.rem(j_a, 16)
            rows_wait(r_a, srow_a, drow_a, sem_a)

            @pl.when(p > 0)
            def _():
                scat_wait(msg_a, sem_sa)

            compute(srow_a, drow_a, msg_a)
            scat_start(msg_a, r_a, sem_sa)
            r_a2 = lax.rem(j_a + 2, 16)
            rows_start(r_a2, srow_a, drow_a, sem_a)
            rows_wait(r_b, srow_b, drow_b, sem_b)

            @pl.when(p > 0)
            def _():
                scat_wait(msg_b, sem_sb)

            compute(srow_b, drow_b, msg_b)
            scat_start(msg_b, r_b, sem_sb)

        scat_wait(msg_a, sem_sa)
        r_last = (NFULL - 1) % 16
        rows_wait(r_last, srow_a, drow_a, sem_a)
        compute(srow_a, drow_a, msg_a)
        scat_start(msg_a, r_last, sem_sa)
        scat_wait(msg_a, sem_sa)
        scat_wait(msg_b, sem_sb)

        @pl.when(gwid < REM_CHUNKS)
        def _():
            rc = REM_BASE // CH + gwid
            pltpu.sync_copy(src_hbm.at[pl.ds(rc, 1)], sbig.at[pl.ds(0, 1)])
            pltpu.sync_copy(dst_hbm.at[pl.ds(rc, 1)], dbig.at[pl.ds(0, 1)])
            rows_start(0, srow_b, drow_b, sem_b)
            rows_wait(0, srow_b, drow_b, sem_b)
            compute(srow_b, drow_b, msg_b)
            pltpu.sync_copy(msg_b, acc_sh.at[dbig.at[0]], add=True)

        plsc.subcore_barrier()

        @pl.when(s < BIG_TILES)
        def _():
            rs = pl.multiple_of(s * ROWS_BIG, 8)
            pltpu.sync_copy(acc_sh.at[pl.ds(rs, ROWS_BIG)],
                            acc_hbm.at[pl.ds(c * N + rs, ROWS_BIG)])

        @pl.when(s >= BIG_TILES)
        def _():
            rs = pl.multiple_of(
                BIG_TILES * ROWS_BIG + (s - BIG_TILES) * ROWS_SMALL, 8)
            pltpu.sync_copy(acc_sh.at[pl.ds(rs, ROWS_SMALL)],
                            acc_hbm.at[pl.ds(c * N + rs, ROWS_SMALL)])

        pltpu.sync_copy(z_v, z_hbm.at[pl.ds(pl.multiple_of(gwid * 16, 8), 16)])

    return k(y, src, dst)


def _combine_body(a0_ref, a1_ref, z_ref, o_ref):
    zsum = jnp.sum(z_ref[...])
    o_ref[...] = jnp.maximum((a0_ref[...] + a1_ref[...]) / zsum, 0.0)


def _combine(acc, z):
    nblk = N // MM_BLK
    return pl.pallas_call(
        _combine_body,
        grid=(nblk,),
        in_specs=[
            pl.BlockSpec((MM_BLK, D), lambda i: (i, 0)),
            pl.BlockSpec((MM_BLK, D), lambda i, _n=nblk: (i + _n, 0)),
            pl.BlockSpec((NW, 16), lambda i: (0, 0)),
        ],
        out_specs=pl.BlockSpec((MM_BLK, D), lambda i: (i, 0)),
        out_shape=jax.ShapeDtypeStruct((N, D), jnp.float32),
    )(acc, acc, z)


def kernel(user_emb, entity_emb, W, W_r, user_indices, item_indices,
           edge_index_ui, edge_index_kg, edge_type_kg):
    tab = jnp.concatenate([user_emb, entity_emb], axis=0)
    t = _transform(tab, W)
    idx = jnp.concatenate([user_indices.astype(jnp.int32),
                           item_indices.astype(jnp.int32) + N_U])
    pad = jnp.zeros((NCH_PAD * CH - E,), jnp.int32)
    src = jnp.concatenate([edge_index_ui[0].astype(jnp.int32), pad])
    dst = jnp.concatenate([edge_index_ui[1].astype(jnp.int32), pad])
    y = _sc_node_gather(t, idx)
    acc, z = _sc_edge_pass(y, src.reshape(NCH_PAD, CH),
                           dst.reshape(NCH_PAD, CH))
    x = _combine(acc, z.reshape(NW, 16))
    return (x[:N_U], x[N_U:])
